# Initial kernel scaffold; baseline (speedup 1.0000x reference)
import jax
import jax.numpy as jnp
from jax.experimental import pallas as pl


def kernel(x, edge_index, edge_attr, batch, We1, be1, W1, b1, We2, be2, W2, b2, We3, be3, W3, b3, Wlin, blin):
    raise NotImplementedError("write your pallas kernel here")


# speedup vs baseline: 2.0573x; 2.0573x over previous
import functools

import jax
import jax.numpy as jnp
from jax import lax
from jax.experimental import pallas as pl
from jax.experimental.pallas import tpu as pltpu
from jax.experimental.pallas import tpu_sc as plsc

N = 10000
E = 320000
IN = 128
HID = 128
ED = 16
G = 64

NC = 2
NS = 16
NW = NC * NS
CHUNK = 128
CPT = -(-E // (NW * CHUNK))
E_PAD = NW * CPT * CHUNK
N_STRIPE = 640
N_PAD = NS * N_STRIPE


def _ea_body(a_ref, w_ref, b_ref, o_ref):
    o_ref[0] = (
        jnp.dot(a_ref[...], w_ref[0], preferred_element_type=jnp.float32)
        + b_ref[0]
    )


def _ea_call(ea_pad, w_cat, b_cat):
    be = 1024
    grid = (3, E_PAD // be)
    return pl.pallas_call(
        _ea_body,
        grid=grid,
        in_specs=[
            pl.BlockSpec((be, ED), lambda l, e: (e, 0)),
            pl.BlockSpec((1, ED, HID), lambda l, e: (l, 0, 0)),
            pl.BlockSpec((1, 1, HID), lambda l, e: (l, 0, 0)),
        ],
        out_specs=pl.BlockSpec((1, be, HID), lambda l, e: (l, e, 0)),
        out_shape=jax.ShapeDtypeStruct((3, E_PAD, HID), jnp.float32),
    )(ea_pad, w_cat, b_cat)


def _sc_body(layer, h_hbm, ea_hbm, idx_hbm, out_hbm,
             idx_v, hbuf, eabuf, agg, sem1, sem2):
    c = lax.axis_index("c")
    s = lax.axis_index("s")
    wid = c * NS + s

    @pl.loop(0, CHUNK)
    def _zrow(r):
        for k in range(HID // 16):
            eabuf[r, pl.ds(k * 16, 16)] = jnp.zeros((16,), jnp.float32)

    @pl.loop(0, N_STRIPE // CHUNK)
    def _zcp(j):
        pltpu.sync_copy(eabuf, agg.at[pl.ds(s * N_STRIPE + j * CHUNK, CHUNK)])

    plsc.subcore_barrier()

    @pl.loop(0, CPT)
    def _edge(cidx):
        base = (wid * CPT + cidx) * CHUNK
        pltpu.sync_copy(idx_hbm.at[wid, cidx], idx_v)
        cp1 = pltpu.async_copy(h_hbm.at[idx_v.at[0]], hbuf, sem1)
        cp2 = pltpu.async_copy(ea_hbm.at[layer, pl.ds(base, CHUNK)], eabuf, sem2)
        cp1.wait()
        cp2.wait()

        @pl.loop(0, CHUNK)
        def _row(r):
            for k in range(HID // 16):
                sl = pl.ds(k * 16, 16)
                hbuf[r, sl] = jnp.maximum(hbuf[r, sl] + eabuf[r, sl], 0.0)

        pltpu.sync_copy(hbuf, agg.at[idx_v.at[1]], add=True)

    plsc.subcore_barrier()

    @pl.loop(0, N_STRIPE // CHUNK)
    def _drain(j):
        row0 = s * N_STRIPE + j * CHUNK
        pltpu.sync_copy(agg.at[pl.ds(row0, CHUNK)],
                        out_hbm.at[c, pl.ds(row0, CHUNK)])


def _sc_call(h, ea_all, layer, idx_p):
    mesh = plsc.VectorSubcoreMesh(core_axis_name="c", subcore_axis_name="s")
    kfn = pl.kernel(
        functools.partial(_sc_body, layer),
        out_type=jax.ShapeDtypeStruct((NC, N_PAD, HID), jnp.float32),
        mesh=mesh,
        scratch_types=[
            pltpu.VMEM((2, CHUNK), jnp.int32),
            pltpu.VMEM((CHUNK, HID), jnp.float32),
            pltpu.VMEM((CHUNK, HID), jnp.float32),
            pltpu.VMEM_SHARED((N_PAD, HID), jnp.float32),
            pltpu.SemaphoreType.DMA,
            pltpu.SemaphoreType.DMA,
        ],
    )
    return kfn(h, ea_all, idx_p)


def _dense_body(h_ref, a_ref, w_ref, b_ref, o_ref):
    t = h_ref[...] + a_ref[0, :N, :] + a_ref[1, :N, :]
    o_ref[...] = jnp.maximum(
        jnp.dot(t, w_ref[...], preferred_element_type=jnp.float32)
        + b_ref[...],
        0.0,
    )


def _dense_call(h, agg, w, b):
    return pl.pallas_call(
        _dense_body,
        out_shape=jax.ShapeDtypeStruct((N, HID), jnp.float32),
    )(h, agg, w, b)


def _pool_body(h_ref, batch_ref, w_ref, b_ref, o_ref):
    gid = lax.broadcasted_iota(jnp.int32, (G, 1), 0)
    pt = (batch_ref[...] == gid).astype(jnp.float32)
    sums = jnp.dot(pt, h_ref[...], preferred_element_type=jnp.float32)
    counts = jnp.sum(pt, axis=1, keepdims=True)
    pooled = sums / jnp.maximum(counts, 1.0)
    o_ref[...] = (
        jnp.dot(pooled, w_ref[...], preferred_element_type=jnp.float32)
        + b_ref[...]
    )


def _pool_call(h, batch2d, w, b):
    return pl.pallas_call(
        _pool_body,
        out_shape=jax.ShapeDtypeStruct((G, HID), jnp.float32),
    )(h, batch2d, w, b)


def kernel(x, edge_index, edge_attr, batch,
           We1, be1, W1, b1,
           We2, be2, W2, b2,
           We3, be3, W3, b3,
           Wlin, blin):
    pad = E_PAD - E
    src = edge_index[0].astype(jnp.int32)
    dst = edge_index[1].astype(jnp.int32)
    src_p = jnp.concatenate([src, jnp.zeros((pad,), jnp.int32)]) \
        .reshape(NW, CPT, CHUNK)
    dst_p = jnp.concatenate([dst, jnp.full((pad,), N, jnp.int32)]) \
        .reshape(NW, CPT, CHUNK)
    idx_p = jnp.stack([src_p, dst_p], axis=2)
    ea_pad = jnp.concatenate(
        [edge_attr, jnp.zeros((pad, ED), jnp.float32)], axis=0)
    w_cat = jnp.stack([We1, We2, We3])
    b_cat = jnp.stack([be1, be2, be3]).reshape(3, 1, HID)

    ea_all = _ea_call(ea_pad, w_cat, b_cat)

    h = x
    for layer, (w, b) in enumerate(((W1, b1), (W2, b2), (W3, b3))):
        agg = _sc_call(h, ea_all, layer, idx_p)
        h = _dense_call(h, agg, w, b.reshape(1, HID))

    return _pool_call(h, batch.astype(jnp.int32).reshape(1, N),
                      Wlin, blin.reshape(1, HID))
